# baseline (speedup 1.0000x reference)
import functools
import jax
import jax.numpy as jnp
from jax import lax
from jax.experimental import pallas as pl
from jax.experimental.pallas import tpu as pltpu
from jax.experimental.pallas import tpu_sc as plsc

_DIM = 128
_BASE = 10000.0

_ROW = 2 * 16 * _DIM
_NW = 32
_P = 8
_RING = 3
_CH = _P * _ROW
_CS_ROW = 2 * _DIM
_CS_CH = _P * _CS_ROW
_SEQ_BLK = 256


def _freq_tables(sq):
    freqs = 1.0 / (_BASE ** (jnp.arange(0, _DIM, 2)[: _DIM // 2].astype(jnp.float32) / _DIM))
    t = jnp.arange(sq).astype(jnp.float32)
    f = jnp.outer(t, freqs)
    return jnp.cos(f), jnp.sin(f)


@functools.lru_cache(maxsize=None)
def _sc_cs_table(sq):
    cos, sin = _freq_tables(sq)
    c_full = jnp.repeat(cos, 2, axis=1)
    s_full = jnp.stack([-sin, sin], axis=-1).reshape(sq, _DIM)
    return jnp.concatenate([c_full, s_full], axis=1).reshape(-1)


@functools.lru_cache(maxsize=None)
def _tc_tables(sq):
    cos, sin = _freq_tables(sq)
    zeros = jnp.zeros_like(sin)
    c_full = jnp.repeat(cos, 2, axis=1)
    a_full = jnp.stack([-sin, zeros], axis=-1).reshape(sq, _DIM)
    b_full = jnp.stack([zeros, sin], axis=-1).reshape(sq, _DIM)
    return (c_full.reshape(sq, 1, _DIM), a_full.reshape(sq, 1, _DIM),
            b_full.reshape(sq, 1, _DIM))


def _tc_body(c_ref, a_ref, b_ref, x_ref, o_ref):
    c = c_ref[...]
    a = a_ref[...]
    b = b_ref[...]
    x = x_ref[...]
    lo = pltpu.roll(x, _DIM - 1, axis=2)
    hi = pltpu.roll(x, 1, axis=2)
    o_ref[...] = x * c + lo * a + hi * b


def _tc_rope(x):
    sq, bsz, nh, hh = x.shape
    c_t, a_t, b_t = _tc_tables(sq)
    fl = bsz * nh
    x3 = x.reshape(sq, fl, hh)
    blk = _SEQ_BLK if sq % _SEQ_BLK == 0 else sq
    tab_spec = pl.BlockSpec((blk, 1, hh), lambda i: (i, 0, 0))
    dat_spec = pl.BlockSpec((blk, fl, hh), lambda i: (i, 0, 0))
    out = pl.pallas_call(
        _tc_body,
        grid=(sq // blk,),
        in_specs=[tab_spec, tab_spec, tab_spec, dat_spec],
        out_specs=dat_spec,
        out_shape=jax.ShapeDtypeStruct((sq, fl, hh), x.dtype),
    )(c_t, a_t, b_t, x3)
    return out.reshape(x.shape)


def _make_sc_rope(sq):
    pos_per_w = sq // _NW
    chunks = pos_per_w // _P
    n = sq * _ROW
    mesh = plsc.VectorSubcoreMesh(core_axis_name="c", subcore_axis_name="s")

    @functools.partial(
        pl.kernel,
        mesh=mesh,
        out_type=jax.ShapeDtypeStruct((n,), jnp.float32),
        scratch_types=[pltpu.VMEM((_CH,), jnp.float32)] * _RING
        + [pltpu.VMEM((_CS_CH,), jnp.float32)] * _RING
        + [pltpu.SemaphoreType.DMA] * (2 * _RING),
    )
    def rope_sc(x_hbm, cs_hbm, o_hbm, *scratch):
        bufs = scratch[:_RING]
        csbufs = scratch[_RING:2 * _RING]
        in_sems = scratch[2 * _RING:3 * _RING]
        out_sems = scratch[3 * _RING:4 * _RING]
        wid = lax.axis_index("s") * 2 + lax.axis_index("c")
        start_pos = wid * pos_per_w
        swp = jnp.bitwise_xor(lax.iota(jnp.int32, 16), 1)
        swp_idx = swp.reshape(16, 1)
        gather_dnums = lax.GatherDimensionNumbers(
            offset_dims=(), collapsed_slice_dims=(0,), start_index_map=(0,))

        def pair_swap(v):
            return lax.gather(
                v, swp_idx, gather_dnums, (1,),
                mode=lax.GatherScatterMode.PROMISE_IN_BOUNDS)

        def in_copies(g, b):
            base = (start_pos + g * _P) * _ROW
            cbase = (start_pos + g * _P) * _CS_ROW
            return (
                pltpu.make_async_copy(x_hbm.at[pl.ds(base, _CH)], bufs[b], in_sems[b]),
                pltpu.make_async_copy(cs_hbm.at[pl.ds(cbase, _CS_CH)], csbufs[b], in_sems[b]),
            )

        def out_copies(g, b):
            base = (start_pos + g * _P) * _ROW
            return (
                pltpu.make_async_copy(bufs[b], o_hbm.at[pl.ds(base, _CH)], out_sems[b]),
            )

        def start_in(g, b):
            for c in in_copies(g, b):
                c.start()

        def wait_in(g, b):
            for c in in_copies(g, b):
                c.wait()

        def start_out(g, b):
            for c in out_copies(g, b):
                c.start()

        def wait_out(g, b):
            for c in out_copies(g, b):
                c.wait()

        def compute(b):
            xb = bufs[b]
            csb = csbufs[b]

            @plsc.parallel_loop(0, _P)
            def _pos_loop(pos):
                for t in range(_DIM // 16):
                    cvec = csb[pl.ds(pos * _CS_ROW + t * 16, 16)]
                    svec = csb[pl.ds(pos * _CS_ROW + _DIM + t * 16, 16)]

                    @plsc.parallel_loop(0, _ROW // _DIM, unroll=4)
                    def _h_loop(h, pos=pos, t=t, cvec=cvec, svec=svec):
                        addr = pos * _ROW + h * _DIM + t * 16
                        x = xb[pl.ds(addr, 16)]
                        sw = pair_swap(x)
                        xb[pl.ds(addr, 16)] = x * cvec + sw * svec

        start_in(0, 0)
        start_in(1, 1)

        def outer(i, carry):
            gbase = i * _RING
            for b in range(_RING):
                g = gbase + b

                @pl.when(g < chunks)
                def _():
                    bp = (b - 1) % _RING
                    wait_in(g, b)
                    compute(b)
                    start_out(g, b)

                    @pl.when(g >= 1)
                    def _():
                        wait_out(g - 1, bp)

                    @pl.when(g + 2 < chunks)
                    def _():
                        start_in(g + 2, bp)

            return carry

        n_outer = (chunks + _RING - 1) // _RING
        lax.fori_loop(0, n_outer, outer, 0)
        wait_out(chunks - 1, (chunks - 1) % _RING)

    return rope_sc


def kernel(query, key):
    sq, bsz, nh, hh = query.shape
    n = sq * bsz * nh * hh
    ko = _make_sc_rope(sq)(key.reshape(n), _sc_cs_table(sq)).reshape(key.shape)
    qo = _tc_rope(query)
    return qo, ko


# speedup vs baseline: 1.6481x; 1.6481x over previous
import functools
import jax
import jax.numpy as jnp
from jax.experimental import pallas as pl

_DIM = 128
_BASE = 10000.0
_SEQ_BLK = 256


@functools.lru_cache(maxsize=None)
def _tables(sq):
    freqs = 1.0 / (_BASE ** (jnp.arange(0, _DIM, 2)[: _DIM // 2].astype(jnp.float32) / _DIM))
    t = jnp.arange(sq).astype(jnp.float32)
    f = jnp.outer(t, freqs)
    cos = jnp.cos(f)
    sin = jnp.sin(f)
    c_full = jnp.repeat(cos, 2, axis=1)
    s_full = jnp.stack([-sin, sin], axis=-1).reshape(sq, _DIM)
    i = jnp.arange(_DIM)
    perm = (i[:, None] ^ 1) == i[None, :]
    p = perm.astype(jnp.float32)
    return (c_full.reshape(sq, 1, _DIM), s_full.reshape(sq, 1, _DIM), p)


def _rope_body(c_ref, s_ref, p_ref, q_ref, k_ref, qo_ref, ko_ref):
    c = c_ref[...]
    s = s_ref[...]
    p = p_ref[...]
    blk, fl, hh = q_ref.shape
    for x_ref, o_ref in ((q_ref, qo_ref), (k_ref, ko_ref)):
        x = x_ref[...]
        sw = jnp.dot(x.reshape(blk * fl, hh), p,
                     preferred_element_type=jnp.float32).reshape(blk, fl, hh)
        o_ref[...] = x * c + sw * s


def kernel(query, key):
    sq, bsz, nh, hh = query.shape
    c_t, s_t, p_t = _tables(sq)
    fl = bsz * nh
    q3 = query.reshape(sq, fl, hh)
    k3 = key.reshape(sq, fl, hh)

    blk = _SEQ_BLK if sq % _SEQ_BLK == 0 else sq
    grid = (sq // blk,)
    tab_spec = pl.BlockSpec((blk, 1, hh), lambda i: (i, 0, 0))
    p_spec = pl.BlockSpec((hh, hh), lambda i: (0, 0))
    dat_spec = pl.BlockSpec((blk, fl, hh), lambda i: (i, 0, 0))

    qo, ko = pl.pallas_call(
        _rope_body,
        grid=grid,
        in_specs=[tab_spec, tab_spec, p_spec, dat_spec, dat_spec],
        out_specs=[dat_spec, dat_spec],
        out_shape=[
            jax.ShapeDtypeStruct((sq, fl, hh), query.dtype),
            jax.ShapeDtypeStruct((sq, fl, hh), key.dtype),
        ],
    )(c_t, s_t, p_t, q3, k3)
    return qo.reshape(query.shape), ko.reshape(key.shape)
